# baseline (speedup 1.0000x reference)
import jax
import jax.numpy as jnp
from jax import lax
from jax.experimental import pallas as pl
from jax.experimental.pallas import tpu as pltpu
from jax.experimental.pallas import tpu_sc as plsc

N = 10000
D = 128
E = 320000

NC = 2
NS = 16
NW = NC * NS
E_PER_W = E // NW
BLK = 128
EDG = 125
NBLK = E_PER_W // EDG
NBLK_PAD = NBLK + 4
ROWS_PER_SUB = 640
N_PAD = NS * ROWS_PER_SUB
INIT_CHUNKS = ROWS_PER_SUB // BLK
X_ZERO = N
DUMMY_ROW = N_PAD - 1


def _sc_body(x_hbm, e4_hbm, zrows_hbm, ones_hbm,
             pagg_hbm, pdeg_hbm,
             ib0, ib1, ib2, ib3, rows0, rows1, agg_sh,
             semg0, semg1, sems0, sems1, semi0, semi1, semi2, semi3):
    c = lax.axis_index("c")
    s = lax.axis_index("s")
    wid = s * NC + c
    r0 = s * ROWS_PER_SUB
    ibs = (ib0, ib1, ib2, ib3)
    semis = (semi0, semi1, semi2, semi3)

    def ib_load(j, blk):
        pltpu.async_copy(e4_hbm.at[wid, blk], ibs[j], semis[j])

    def ib_wait(j):
        pltpu.make_async_copy(e4_hbm.at[wid, 0], ibs[j], semis[j]).wait()

    def row_wait(rows, sem):
        pltpu.make_async_copy(zrows_hbm, rows, sem).wait()

    def zero_acc():
        pltpu.sync_copy(zrows_hbm, rows0)
        for k in range(INIT_CHUNKS):
            pltpu.sync_copy(rows0, agg_sh.at[pl.ds(r0 + k * BLK, BLK)])

    def write_out(out_hbm):
        for k in range(INIT_CHUNKS):
            row = r0 + k * BLK
            pltpu.sync_copy(agg_sh.at[pl.ds(row, BLK)], rows0)
            pltpu.sync_copy(rows0, out_hbm.at[c, pl.ds(row, BLK)])

    pltpu.sync_copy(ones_hbm, rows1)
    zero_acc()
    plsc.subcore_barrier()
    ib_load(0, 0)
    ib_load(1, 1)

    def deg_pair(p, carry):
        a = 2 * p
        ib_wait(0)
        pltpu.sync_copy(rows1, agg_sh.at[ib0.at[1]], add=True)
        ib_load(0, a + 2)
        ib_wait(1)
        pltpu.sync_copy(rows1, agg_sh.at[ib1.at[1]], add=True)
        ib_load(1, a + 3)
        return carry

    lax.fori_loop(0, NBLK // 2, deg_pair, 0)
    ib_wait(0)
    ib_wait(1)
    plsc.subcore_barrier()
    write_out(pdeg_hbm)
    zero_acc()
    plsc.subcore_barrier()

    for j in range(4):
        ib_load(j, j)
    ib_wait(0)
    pltpu.async_copy(x_hbm.at[ib0.at[0]], rows0, semg0)
    ib_wait(1)
    pltpu.async_copy(x_hbm.at[ib1.at[0]], rows1, semg1)

    def quad_body(q, carry):
        m = 4 * q
        for j in range(4):
            rows, semg = (rows0, semg0) if j % 2 == 0 else (rows1, semg1)
            jn = (j + 2) % 4
            row_wait(rows, semg)
            pltpu.sync_copy(rows, agg_sh.at[ibs[j].at[1]], add=True)
            ib_wait(jn)
            pltpu.async_copy(x_hbm.at[ibs[jn].at[0]], rows, semg)
            ib_load(j, m + j + 4)
        return carry

    lax.fori_loop(0, NBLK // 4, quad_body, 0)
    row_wait(rows0, semg0)
    row_wait(rows1, semg1)
    ib_wait(2)
    ib_wait(3)
    plsc.subcore_barrier()
    write_out(pagg_hbm)


_sc_scatter = pl.kernel(
    _sc_body,
    out_type=[
        jax.ShapeDtypeStruct((NC, N_PAD, D), jnp.float32),
        jax.ShapeDtypeStruct((NC, N_PAD, D), jnp.float32),
    ],
    mesh=plsc.VectorSubcoreMesh(core_axis_name="c", subcore_axis_name="s"),
    scratch_types=[
        pltpu.VMEM((2, BLK), jnp.int32),
        pltpu.VMEM((2, BLK), jnp.int32),
        pltpu.VMEM((2, BLK), jnp.int32),
        pltpu.VMEM((2, BLK), jnp.int32),
        pltpu.VMEM((BLK, D), jnp.float32),
        pltpu.VMEM((BLK, D), jnp.float32),
        pltpu.VMEM_SHARED((N_PAD, D), jnp.float32),
        pltpu.SemaphoreType.DMA,
        pltpu.SemaphoreType.DMA,
        pltpu.SemaphoreType.DMA,
        pltpu.SemaphoreType.DMA,
        pltpu.SemaphoreType.DMA,
        pltpu.SemaphoreType.DMA,
        pltpu.SemaphoreType.DMA,
        pltpu.SemaphoreType.DMA,
    ],
)


ROWS_PER_STEP = 1000
GRID = N // ROWS_PER_STEP


def _tc_body(pagg0, pagg1, pdeg0, pdeg1, x, wm, ws, out):
    i = pl.program_id(0)
    agg = pagg0[0] + pagg1[0]
    deg = pdeg0[0] + pdeg1[0]
    aggn = agg / jnp.maximum(deg, 1.0)
    h = jnp.dot(aggn, wm[...], preferred_element_type=jnp.float32)
    h = h + jnp.dot(x[...], ws[...], preferred_element_type=jnp.float32)
    h = jnp.maximum(h, 0.0)
    part = jnp.sum(h, axis=0, keepdims=True) * jnp.float32(1.0 / N)

    @pl.when(i == 0)
    def _():
        out[...] = part

    @pl.when(i > 0)
    def _():
        out[...] += part


_tc_combine = pl.pallas_call(
    _tc_body,
    grid=(GRID,),
    in_specs=[
        pl.BlockSpec((1, ROWS_PER_STEP, D), lambda i: (0, i, 0)),
        pl.BlockSpec((1, ROWS_PER_STEP, D), lambda i: (1, i, 0)),
        pl.BlockSpec((1, ROWS_PER_STEP, D), lambda i: (0, i, 0)),
        pl.BlockSpec((1, ROWS_PER_STEP, D), lambda i: (1, i, 0)),
        pl.BlockSpec((ROWS_PER_STEP, D), lambda i: (i, 0)),
        pl.BlockSpec((D, D), lambda i: (0, 0)),
        pl.BlockSpec((D, D), lambda i: (0, 0)),
    ],
    out_specs=pl.BlockSpec((1, D), lambda i: (0, 0)),
    out_shape=jax.ShapeDtypeStruct((1, D), jnp.float32),
)


@jax.jit
def kernel(x, edge_index, W_msg, W_self):
    pad_src = jnp.broadcast_to(jnp.arange(BLK - EDG, dtype=jnp.int32),
                               (NW, NBLK, BLK - EDG))
    src_p = jnp.concatenate(
        [edge_index[0].reshape(NW, NBLK, EDG), pad_src], axis=2)
    src_p = jnp.concatenate(
        [src_p, jnp.broadcast_to(jnp.arange(BLK, dtype=jnp.int32),
                                 (NW, NBLK_PAD - NBLK, BLK))], axis=1)
    dummy = jnp.broadcast_to(
        (N + 3 * jnp.arange(NW, dtype=jnp.int32))[:, None, None]
        + jnp.arange(BLK - EDG, dtype=jnp.int32)[None, None, :],
        (NW, NBLK, BLK - EDG))
    dst_p = jnp.concatenate(
        [edge_index[1].reshape(NW, NBLK, EDG), dummy], axis=2)
    dst_p = jnp.pad(dst_p, ((0, 0), (0, NBLK_PAD - NBLK), (0, 0)),
                    constant_values=DUMMY_ROW)
    e4 = jnp.stack([src_p, dst_p], axis=2)
    zrows = jnp.zeros((BLK, D), jnp.float32)
    ones = jnp.ones((BLK, D), jnp.float32)
    pagg, pdeg = _sc_scatter(x, e4, zrows, ones)
    pooled = _tc_combine(pagg, pagg, pdeg, pdeg, x, W_msg, W_self)
    return pooled.reshape(D)


# speedup vs baseline: 1.2606x; 1.2606x over previous
import jax
import jax.numpy as jnp
from jax import lax
from jax.experimental import pallas as pl
from jax.experimental.pallas import tpu as pltpu
from jax.experimental.pallas import tpu_sc as plsc

N = 10000
D = 128
E = 320000

NC = 2
NS = 16
NW = NC * NS
E_PER_W = E // NW
BLK = 80
EDG = 80
NBLK = E_PER_W // EDG
NBLK_PAD = 128
ROWS_PER_SUB = 640
N_PAD = NS * ROWS_PER_SUB
INIT_CHUNKS = ROWS_PER_SUB // BLK
X_ZERO = N
DUMMY_ROW = N_PAD - 1


def _sc_body(x_hbm, e4_hbm, zrows_hbm, ones_hbm,
             pagg_hbm, pdeg_hbm,
             ib0, ib1, ib2, ib3, rows0, rows1, agg_sh,
             semg0, semg1, sems0, sems1, semi0, semi1, semi2, semi3):
    c = lax.axis_index("c")
    s = lax.axis_index("s")
    wid = s * NC + c
    r0 = s * ROWS_PER_SUB
    ibs = (ib0, ib1, ib2, ib3)
    semis = (semi0, semi1, semi2, semi3)

    def ib_load(j, blk):
        pltpu.async_copy(e4_hbm.at[wid, blk], ibs[j], semis[j])

    def ib_wait(j):
        pltpu.make_async_copy(e4_hbm.at[wid, 0], ibs[j], semis[j]).wait()

    def row_wait(rows, sem):
        pltpu.make_async_copy(zrows_hbm, rows, sem).wait()

    def zero_acc():
        pltpu.sync_copy(zrows_hbm, rows0)
        for k in range(INIT_CHUNKS):
            pltpu.sync_copy(rows0, agg_sh.at[pl.ds(r0 + k * BLK, BLK)])

    def write_out(out_hbm):
        for k in range(INIT_CHUNKS):
            row = r0 + k * BLK
            pltpu.sync_copy(agg_sh.at[pl.ds(row, BLK)], rows0)
            pltpu.sync_copy(rows0, out_hbm.at[c, pl.ds(row, BLK)])

    pltpu.sync_copy(ones_hbm, rows1)
    zero_acc()
    plsc.subcore_barrier()
    ib_load(0, 0)
    ib_load(1, 1)

    def deg_pair(p, carry):
        a = 2 * p
        ib_wait(0)
        pltpu.sync_copy(rows1, agg_sh.at[ib0.at[1]], add=True)
        ib_load(0, a + 2)
        ib_wait(1)
        pltpu.sync_copy(rows1, agg_sh.at[ib1.at[1]], add=True)
        ib_load(1, a + 3)
        return carry

    lax.fori_loop(0, NBLK // 2, deg_pair, 0)
    ib_wait(0)
    pltpu.sync_copy(rows1, agg_sh.at[ib0.at[1]], add=True)
    ib_wait(1)
    plsc.subcore_barrier()
    write_out(pdeg_hbm)
    zero_acc()
    plsc.subcore_barrier()

    for j in range(4):
        ib_load(j, j)
    ib_wait(0)
    pltpu.async_copy(x_hbm.at[ib0.at[0]], rows0, semg0)
    ib_wait(1)
    pltpu.async_copy(x_hbm.at[ib1.at[0]], rows1, semg1)

    def quad_body(q, carry):
        m = 4 * q
        for j in range(4):
            rows, semg = (rows0, semg0) if j % 2 == 0 else (rows1, semg1)
            jn = (j + 2) % 4
            row_wait(rows, semg)
            pltpu.sync_copy(rows, agg_sh.at[ibs[j].at[1]], add=True)
            ib_wait(jn)
            pltpu.async_copy(x_hbm.at[ibs[jn].at[0]], rows, semg)
            ib_load(j, m + j + 4)
        return carry

    lax.fori_loop(0, NBLK // 4, quad_body, 0)
    row_wait(rows0, semg0)
    pltpu.sync_copy(rows0, agg_sh.at[ib0.at[1]], add=True)
    row_wait(rows1, semg1)
    ib_wait(2)
    ib_wait(3)
    plsc.subcore_barrier()
    write_out(pagg_hbm)


_sc_scatter = pl.kernel(
    _sc_body,
    out_type=[
        jax.ShapeDtypeStruct((NC, N_PAD, D), jnp.float32),
        jax.ShapeDtypeStruct((NC, N_PAD, D), jnp.float32),
    ],
    mesh=plsc.VectorSubcoreMesh(core_axis_name="c", subcore_axis_name="s"),
    scratch_types=[
        pltpu.VMEM((2, BLK), jnp.int32),
        pltpu.VMEM((2, BLK), jnp.int32),
        pltpu.VMEM((2, BLK), jnp.int32),
        pltpu.VMEM((2, BLK), jnp.int32),
        pltpu.VMEM((BLK, D), jnp.float32),
        pltpu.VMEM((BLK, D), jnp.float32),
        pltpu.VMEM_SHARED((N_PAD, D), jnp.float32),
        pltpu.SemaphoreType.DMA,
        pltpu.SemaphoreType.DMA,
        pltpu.SemaphoreType.DMA,
        pltpu.SemaphoreType.DMA,
        pltpu.SemaphoreType.DMA,
        pltpu.SemaphoreType.DMA,
        pltpu.SemaphoreType.DMA,
        pltpu.SemaphoreType.DMA,
    ],
)


ROWS_PER_STEP = 1000
GRID = N // ROWS_PER_STEP


def _tc_body(pagg0, pagg1, pdeg0, pdeg1, x, wm, ws, out):
    i = pl.program_id(0)
    agg = pagg0[0] + pagg1[0]
    deg = pdeg0[0] + pdeg1[0]
    aggn = agg / jnp.maximum(deg, 1.0)
    h = jnp.dot(aggn, wm[...], preferred_element_type=jnp.float32)
    h = h + jnp.dot(x[...], ws[...], preferred_element_type=jnp.float32)
    h = jnp.maximum(h, 0.0)
    part = jnp.sum(h, axis=0, keepdims=True) * jnp.float32(1.0 / N)

    @pl.when(i == 0)
    def _():
        out[...] = part

    @pl.when(i > 0)
    def _():
        out[...] += part


_tc_combine = pl.pallas_call(
    _tc_body,
    grid=(GRID,),
    in_specs=[
        pl.BlockSpec((1, ROWS_PER_STEP, D), lambda i: (0, i, 0)),
        pl.BlockSpec((1, ROWS_PER_STEP, D), lambda i: (1, i, 0)),
        pl.BlockSpec((1, ROWS_PER_STEP, D), lambda i: (0, i, 0)),
        pl.BlockSpec((1, ROWS_PER_STEP, D), lambda i: (1, i, 0)),
        pl.BlockSpec((ROWS_PER_STEP, D), lambda i: (i, 0)),
        pl.BlockSpec((D, D), lambda i: (0, 0)),
        pl.BlockSpec((D, D), lambda i: (0, 0)),
    ],
    out_specs=pl.BlockSpec((1, D), lambda i: (0, 0)),
    out_shape=jax.ShapeDtypeStruct((1, D), jnp.float32),
)


@jax.jit
def kernel(x, edge_index, W_msg, W_self):
    src_p = jnp.concatenate(
        [edge_index[0].reshape(NW, NBLK, EDG),
         jnp.broadcast_to(jnp.arange(BLK, dtype=jnp.int32),
                          (NW, NBLK_PAD - NBLK, BLK))], axis=1)
    dst_p = jnp.pad(edge_index[1].reshape(NW, NBLK, EDG),
                    ((0, 0), (0, NBLK_PAD - NBLK), (0, 0)),
                    constant_values=DUMMY_ROW)
    e4 = jnp.stack([src_p, dst_p], axis=2)
    zrows = jnp.zeros((BLK, D), jnp.float32)
    ones = jnp.ones((BLK, D), jnp.float32)
    pagg, pdeg = _sc_scatter(x, e4, zrows, ones)
    pooled = _tc_combine(pagg, pagg, pdeg, pdeg, x, W_msg, W_self)
    return pooled.reshape(D)
